# Initial kernel scaffold; baseline (speedup 1.0000x reference)
import jax
import jax.numpy as jnp
from jax.experimental import pallas as pl


def kernel(y_true_cls, y_pred_cls, y_true_geo, y_pred_geo, training_mask):
    raise NotImplementedError("write your pallas kernel here")


# speedup vs baseline: 4.9876x; 4.9876x over previous
import jax
import jax.numpy as jnp
from jax import lax
from jax.experimental import pallas as pl
from jax.experimental.pallas import tpu as pltpu

_I32_MIN = -2147483648


def _monotone_key(x):
    b = lax.bitcast_convert_type(x, jnp.int32)
    return b ^ (lax.shift_right_arithmetic(b, 31) & jnp.int32(0x7FFFFFFF))


def _kth_largest_keys(key_a, k_a, key_b, k_b):
    def body(_, carry):
        pa, pb, bit = carry
        ta = pa | bit
        tb = pb | bit
        ca = jnp.sum((key_a >= (ta ^ _I32_MIN)).astype(jnp.int32))
        cb = jnp.sum((key_b >= (tb ^ _I32_MIN)).astype(jnp.int32))
        pa = jnp.where(ca >= k_a, ta, pa)
        pb = jnp.where(cb >= k_b, tb, pb)
        return pa, pb, lax.shift_right_logical(bit, 1)

    pa, pb, _ = lax.fori_loop(
        0, 32, body, (jnp.int32(0), jnp.int32(0), jnp.int32(_I32_MIN))
    )
    return pa ^ _I32_MIN, pb ^ _I32_MIN


def _loss_body(gt_ref, sc_ref, tm_ref, ytg_ref, ypg_ref, out_ref, acc_ref):
    i = pl.program_id(0)
    nb = pl.num_programs(0)

    gt = gt_ref[0]
    sc = sc_ref[0]
    tm = tm_ref[0]
    n = gt.shape[0] * gt.shape[1]

    skey = _monotone_key(sc)
    pos = gt > 0.5
    neg = gt < 0.5
    tmpos = tm > 0.5
    pos_num = jnp.sum((pos & tmpos).astype(jnp.int32))
    neg_full = jnp.sum(neg.astype(jnp.int32))
    neg_half = lax.shift_right_arithmetic(neg_full, 1)
    idx_a = neg_half - 1
    idx_a = jnp.where(idx_a < 0, idx_a + n, idx_a)
    k_a = idx_a + 1
    neg_num = jnp.minimum(pos_num * 3, neg_full)
    k_b = jnp.maximum(neg_num, 1)
    mkey = jnp.where(neg, skey, _I32_MIN)

    thr_a, thr_b = _kth_largest_keys(skey, k_a, mkey, k_b)

    mask_a = (skey >= thr_a).astype(jnp.float32)
    mask_b = (((skey >= thr_b) | pos) & tmpos).astype(jnp.float32)
    ohem = jnp.where(pos_num == 0, mask_a, jnp.where(neg_num == 0, tm, mask_b))

    di = jnp.sum(gt * sc * ohem)
    du1 = jnp.sum(gt * ohem)
    du2 = jnp.sum(sc * ohem)

    d1g = ytg_ref[0, 0]
    d2g = ytg_ref[0, 1]
    d3g = ytg_ref[0, 2]
    d4g = ytg_ref[0, 3]
    thg = ytg_ref[0, 4]
    d1p = ypg_ref[0, 0]
    d2p = ypg_ref[0, 1]
    d3p = ypg_ref[0, 2]
    d4p = ypg_ref[0, 3]
    thp = ypg_ref[0, 4]

    area_gt = (d1g + d3g) * (d2g + d4g)
    area_pred = (d1p + d3p) * (d2p + d4p)
    w_union = jnp.minimum(d2g, d2p) + jnp.minimum(d4g, d4p)
    h_union = jnp.minimum(d1g, d1p) + jnp.minimum(d3g, d3p)
    area_i = w_union * h_union
    area_u = area_gt + area_pred - area_i
    l_aabb = -jnp.log((area_i + 1.0) / (area_u + 1.0))
    l_theta = 1.0 - jnp.cos(thp - thg)
    l_g = l_aabb + 20.0 * l_theta
    g_part = jnp.sum(l_g * gt * tm)

    @pl.when(i == 0)
    def _init():
        acc_ref[0] = 0.0
        acc_ref[1] = 0.0
        acc_ref[2] = 0.0
        acc_ref[3] = 0.0

    acc_ref[0] = acc_ref[0] + g_part
    acc_ref[1] = acc_ref[1] + di
    acc_ref[2] = acc_ref[2] + du1
    acc_ref[3] = acc_ref[3] + du2

    @pl.when(i == nb - 1)
    def _fin():
        union = acc_ref[2] + acc_ref[3] + 1e-5
        cls = (1.0 - 2.0 * acc_ref[1] / union) * 0.01
        out_ref[0, 0] = acc_ref[0] / (nb * n) + cls


def kernel(y_true_cls, y_pred_cls, y_true_geo, y_pred_geo, training_mask):
    b, _, h, w = y_true_cls.shape
    gt = y_true_cls.reshape(b, h, w)
    sc = y_pred_cls.reshape(b, h, w)
    tm = training_mask.reshape(b, h, w)

    out = pl.pallas_call(
        _loss_body,
        grid=(b,),
        in_specs=[
            pl.BlockSpec((1, h, w), lambda i: (i, 0, 0)),
            pl.BlockSpec((1, h, w), lambda i: (i, 0, 0)),
            pl.BlockSpec((1, h, w), lambda i: (i, 0, 0)),
            pl.BlockSpec((1, 5, h, w), lambda i: (i, 0, 0, 0)),
            pl.BlockSpec((1, 5, h, w), lambda i: (i, 0, 0, 0)),
        ],
        out_specs=pl.BlockSpec(memory_space=pltpu.SMEM),
        out_shape=jax.ShapeDtypeStruct((1, 1), jnp.float32),
        scratch_shapes=[pltpu.SMEM((4,), jnp.float32)],
    )(gt, sc, tm, y_true_geo, y_pred_geo)
    return out[0, 0]
